# Initial kernel scaffold; baseline (speedup 1.0000x reference)
import jax
import jax.numpy as jnp
from jax.experimental import pallas as pl


def kernel(emb, edge_index):
    raise NotImplementedError("write your pallas kernel here")


# speedup vs baseline: 14.8296x; 14.8296x over previous
import functools

import jax
import jax.numpy as jnp
from jax import lax
from jax.experimental import pallas as pl
from jax.experimental.pallas import tpu as pltpu
from jax.experimental.pallas import tpu_sc as plsc

N_USERS = 50000
N_NODES = 100000
D = 32
DH = 16
E = 1600000
LAYERS = 3

NC, NS = 2, 16
W = 128
K = 8
NP = 100096
EP = 1638400
ROWS_ALL = EP // W
RPT_MAIN = ROWS_ALL // NS
RPT_DEG = ROWS_ALL // (NC * NS)
NODE_RPT = NP // NS
ZB = NODE_RPT // 8

_mesh = plsc.VectorSubcoreMesh(core_axis_name="c", subcore_axis_name="s")
_sc_params = pltpu.CompilerParams(use_tc_tiling_on_sc=False)


def _fill(ref, val, n):
    def body(i, carry):
        ref[i] = jnp.full((DH,), val, jnp.float32)
        return carry
    lax.fori_loop(0, n, body, 0)


def _zero_acc(acc, zbuf, s):
    for k in range(NODE_RPT // ZB):
        pltpu.sync_copy(zbuf.at[pl.ds(0, ZB)],
                        acc.at[pl.ds(s * NODE_RPT + k * ZB, ZB)])


def _copy_out(acc, zbuf, out_hbm, base, s):
    for k in range(NODE_RPT // ZB):
        off = s * NODE_RPT + k * ZB
        pltpu.sync_copy(acc.at[pl.ds(off, ZB)], zbuf.at[pl.ds(0, ZB)])
        pltpu.sync_copy(zbuf.at[pl.ds(0, ZB)], out_hbm.at[pl.ds(base + off, ZB)])


@functools.partial(
    pl.kernel,
    out_type=jax.ShapeDtypeStruct((NC * NP, DH), jnp.float32),
    mesh=_mesh,
    scratch_types=[
        pltpu.VMEM((K, W), jnp.int32),
        pltpu.VMEM((W, DH), jnp.float32),
        pltpu.VMEM((ZB, DH), jnp.float32),
        pltpu.VMEM_SHARED((NP, DH), jnp.float32),
        pltpu.SemaphoreType.DMA,
    ],
    compiler_params=_sc_params,
)
def _deg_kernel(col2d, out, colv, ones, zbuf, acc, sem):
    c = lax.axis_index("c")
    s = lax.axis_index("s")
    _fill(zbuf, 0.0, ZB)
    _fill(ones, 1.0, W)
    _zero_acc(acc, zbuf, s)
    plsc.subcore_barrier()

    tile_base = (c * NS + s) * RPT_DEG

    def chunk(m, carry):
        pltpu.sync_copy(col2d.at[pl.ds(tile_base + m * K, K), :], colv)
        ds = []
        for j in range(K):
            ds.append(pltpu.async_copy(ones, acc.at[colv.at[j]], sem, add=True))
        for d in ds:
            d.wait()
        return carry

    lax.fori_loop(0, RPT_DEG // K, chunk, 0)
    plsc.subcore_barrier()
    _copy_out(acc, zbuf, out, c * NP, s)


@functools.partial(
    pl.kernel,
    out_type=jax.ShapeDtypeStruct((NC * NP, DH), jnp.float32),
    mesh=_mesh,
    scratch_types=[
        pltpu.VMEM((K, W), jnp.int32),
        pltpu.VMEM((K, W), jnp.int32),
        pltpu.VMEM((K * W, DH), jnp.float32),
        pltpu.VMEM_SHARED((NP, DH), jnp.float32),
        pltpu.SemaphoreType.DMA,
        pltpu.SemaphoreType.DMA,
    ],
    compiler_params=_sc_params,
)
def _edge_kernel(rowadj2d, col2d, y, out, rowv, colv, msg, acc, gsem, ssem):
    c = lax.axis_index("c")
    s = lax.axis_index("s")
    _fill(msg, 0.0, ZB)
    _zero_acc(acc, msg, s)
    plsc.subcore_barrier()

    tile_base = c * ROWS_ALL + s * RPT_MAIN
    col_base = s * RPT_MAIN

    def chunk(m, carry):
        pltpu.sync_copy(rowadj2d.at[pl.ds(tile_base + m * K, K), :], rowv)
        pltpu.sync_copy(col2d.at[pl.ds(col_base + m * K, K), :], colv)
        gds = []
        for j in range(K):
            gds.append(pltpu.async_copy(
                y.at[rowv.at[j]], msg.at[pl.ds(j * W, W)], gsem))
        sds = []
        for j in range(K):
            gds[j].wait()
            sds.append(pltpu.async_copy(
                msg.at[pl.ds(j * W, W)], acc.at[colv.at[j]], ssem, add=True))
        for d in sds:
            d.wait()
        return carry

    lax.fori_loop(0, RPT_MAIN // K, chunk, 0)
    plsc.subcore_barrier()
    _copy_out(acc, msg, out, c * NP, s)


_TCROWS = 3128


def _prep_body(emb_ref, dega_ref, degb_ref, dinv_ref, y0_ref, s0_ref):
    deg = dega_ref[...] + degb_ref[...]
    dinv = jnp.where(deg > 0, lax.rsqrt(deg), 0.0)
    dinv_ref[...] = dinv
    half = emb_ref[0]
    y0_ref[0] = half * dinv
    s0_ref[0] = half


def _tc_prep(emb2, deg_flat):
    grid = (NC, NP // _TCROWS)
    return pl.pallas_call(
        _prep_body,
        grid=grid,
        in_specs=[
            pl.BlockSpec((1, _TCROWS, DH), lambda c, i: (c, i, 0)),
            pl.BlockSpec((_TCROWS, DH), lambda c, i: (i, 0)),
            pl.BlockSpec((_TCROWS, DH), lambda c, i: (i + NP // _TCROWS, 0)),
        ],
        out_specs=[
            pl.BlockSpec((_TCROWS, DH), lambda c, i: (i, 0)),
            pl.BlockSpec((1, _TCROWS, DH), lambda c, i: (c, i, 0)),
            pl.BlockSpec((1, _TCROWS, DH), lambda c, i: (c, i, 0)),
        ],
        out_shape=[
            jax.ShapeDtypeStruct((NP, DH), jnp.float32),
            jax.ShapeDtypeStruct((NC, NP, DH), jnp.float32),
            jax.ShapeDtypeStruct((NC, NP, DH), jnp.float32),
        ],
    )(emb2, deg_flat, deg_flat)


def _node_body(acc_ref, dinv_ref, sin_ref, sout_ref, y_ref):
    x = acc_ref[0] * dinv_ref[...]
    sout_ref[0] = sin_ref[0] + x
    y_ref[0] = x * dinv_ref[...]


def _tc_node(acc_flat, dinv_rep, s_in):
    grid = (NC, NP // _TCROWS)
    return pl.pallas_call(
        _node_body,
        grid=grid,
        in_specs=[
            pl.BlockSpec((1, _TCROWS, DH), lambda c, i: (c, i, 0)),
            pl.BlockSpec((_TCROWS, DH), lambda c, i: (i, 0)),
            pl.BlockSpec((1, _TCROWS, DH), lambda c, i: (c, i, 0)),
        ],
        out_specs=[
            pl.BlockSpec((1, _TCROWS, DH), lambda c, i: (c, i, 0)),
            pl.BlockSpec((1, _TCROWS, DH), lambda c, i: (c, i, 0)),
        ],
        out_shape=[
            jax.ShapeDtypeStruct((NC, NP, DH), jnp.float32),
            jax.ShapeDtypeStruct((NC, NP, DH), jnp.float32),
        ],
    )(acc_flat.reshape(NC, NP, DH), dinv_rep, s_in)


def _final_body(s0_ref, s1_ref, out_ref):
    out_ref[:, :DH] = s0_ref[...] * 0.25
    out_ref[:, DH:] = s1_ref[...] * 0.25


def _tc_final(s):
    grid = (NP // _TCROWS,)
    return pl.pallas_call(
        _final_body,
        grid=grid,
        in_specs=[
            pl.BlockSpec((_TCROWS, DH), lambda i: (i, 0)),
            pl.BlockSpec((_TCROWS, DH), lambda i: (i, 0)),
        ],
        out_specs=pl.BlockSpec((_TCROWS, D), lambda i: (i, 0)),
        out_shape=jax.ShapeDtypeStruct((NP, D), jnp.float32),
    )(s[0], s[1])


def kernel(emb, edge_index):
    row = edge_index[0]
    col = edge_index[1]
    pad = EP - E
    row_p = jnp.concatenate([row, jnp.full((pad,), N_NODES, jnp.int32)])
    col_p = jnp.concatenate([col, jnp.full((pad,), N_NODES, jnp.int32)])
    rowadj2d = jnp.concatenate([row_p, row_p + NP]).reshape(2 * ROWS_ALL, W)
    col2d = col_p.reshape(ROWS_ALL, W)
    emb_p = jnp.pad(emb, ((0, NP - N_NODES), (0, 0)))
    emb2 = jnp.stack([emb_p[:, :DH], emb_p[:, DH:]])

    deg_flat = _deg_kernel(col2d)
    dinv_rep, y, s = _tc_prep(emb2, deg_flat)
    for _ in range(LAYERS):
        acc_flat = _edge_kernel(rowadj2d, col2d, y.reshape(NC * NP, DH))
        s, y = _tc_node(acc_flat, dinv_rep, s)
    out_full = _tc_final(s)
    return (out_full[:N_USERS], out_full[N_USERS:N_NODES])
